# baseline (speedup 1.0000x reference)
import functools

import jax
import jax.numpy as jnp
from jax import lax
from jax.experimental import pallas as pl
from jax.experimental.pallas import tpu as pltpu
from jax.experimental.pallas import tpu_sc as plsc

B = 16384
D = 64
L = 16

_info = plsc.get_sparse_core_info()
NC, NS = _info.num_cores, _info.num_subcores
NW = NC * NS
BPW = B // NW
NCHUNK = BPW // 128
NGROUP = BPW // L


def _bpr_body(users_hbm, pos_hbm, neg_hbm, uemb_hbm, iemb_hbm,
              rui_hbm, ruj_hbm, loss_hbm,
              uidx_v, pidx_v, nidx_v, urows_v, prows_v, nrows_v,
              rui_v, ruj_v, loss_v, sem):
    wid = lax.axis_index("s") * NC + lax.axis_index("c")
    pltpu.sync_copy(users_hbm.at[pl.ds(wid * NCHUNK, NCHUNK)], uidx_v)
    pltpu.sync_copy(pos_hbm.at[pl.ds(wid * NCHUNK, NCHUNK)], pidx_v)
    pltpu.sync_copy(neg_hbm.at[pl.ds(wid * NCHUNK, NCHUNK)], nidx_v)

    copies = []
    for j in range(NCHUNK):
        dst = pl.ds(j * 128, 128)
        copies.append(pltpu.async_copy(uemb_hbm.at[uidx_v.at[j]],
                                       urows_v.at[dst], sem))
        copies.append(pltpu.async_copy(iemb_hbm.at[pidx_v.at[j]],
                                       prows_v.at[dst], sem))
        copies.append(pltpu.async_copy(iemb_hbm.at[nidx_v.at[j]],
                                       nrows_v.at[dst], sem))
    for c in copies:
        c.wait()

    iota = lax.broadcasted_iota(jnp.int32, (L,), 0)
    zero = jnp.zeros((L,), jnp.float32)

    def group(g, carry):
        l1, l2, l3 = carry
        rowv = g * L + iota
        rui_acc = zero
        ruj_acc = zero
        for d in range(D):
            colv = jnp.full((L,), d, jnp.int32)
            iu = plsc.load_gather(urows_v, [rowv, colv])
            ip = plsc.load_gather(prows_v, [rowv, colv])
            iv = plsc.load_gather(nrows_v, [rowv, colv])
            rui_acc = rui_acc + iu * ip
            ruj_acc = ruj_acc + iu * iv
            l1 = l1 + iu * iu
            l2 = l2 + ip * ip
            l3 = l3 + iv * iv
        rui_v[pl.ds(g * L, L)] = rui_acc
        ruj_v[pl.ds(g * L, L)] = ruj_acc
        return (l1, l2, l3)

    l1, l2, l3 = lax.fori_loop(0, NGROUP, group, (zero, zero, zero))
    loss_v[...] = l1 + l2 + l3

    pltpu.sync_copy(rui_v, rui_hbm.at[pl.ds(wid * BPW, BPW)])
    pltpu.sync_copy(ruj_v, ruj_hbm.at[pl.ds(wid * BPW, BPW)])
    pltpu.sync_copy(loss_v, loss_hbm.at[wid])


@jax.jit
def _bpr_sc(users, pos_items, neg_items, user_emb, item_emb):
    mesh = plsc.VectorSubcoreMesh(core_axis_name="c", subcore_axis_name="s")
    k = functools.partial(
        pl.kernel,
        mesh=mesh,
        compiler_params=pltpu.CompilerParams(needs_layout_passes=False,
                                              use_tc_tiling_on_sc=False),
        out_type=[
            jax.ShapeDtypeStruct((B,), jnp.float32),
            jax.ShapeDtypeStruct((B,), jnp.float32),
            jax.ShapeDtypeStruct((NW, L), jnp.float32),
        ],
        scratch_types=[
            pltpu.VMEM((NCHUNK, 128), jnp.int32),
            pltpu.VMEM((NCHUNK, 128), jnp.int32),
            pltpu.VMEM((NCHUNK, 128), jnp.int32),
            pltpu.VMEM((BPW, D), jnp.float32),
            pltpu.VMEM((BPW, D), jnp.float32),
            pltpu.VMEM((BPW, D), jnp.float32),
            pltpu.VMEM((BPW,), jnp.float32),
            pltpu.VMEM((BPW,), jnp.float32),
            pltpu.VMEM((L,), jnp.float32),
            pltpu.SemaphoreType.DMA,
        ],
    )(_bpr_body)
    u2 = users.reshape(NW * NCHUNK, 128).astype(jnp.int32)
    p2 = pos_items.reshape(NW * NCHUNK, 128).astype(jnp.int32)
    n2 = neg_items.reshape(NW * NCHUNK, 128).astype(jnp.int32)
    rui, ruj, loss_parts = k(u2, p2, n2, user_emb, item_emb)
    return (rui.reshape(B, 1), ruj.reshape(B, 1), jnp.sum(loss_parts))


def kernel(users, pos_items, neg_items, user_emb, item_emb):
    return _bpr_sc(users, pos_items, neg_items, user_emb, item_emb)


# speedup vs baseline: 1.2905x; 1.2905x over previous
import functools

import jax
import jax.numpy as jnp
from jax import lax
from jax.experimental import pallas as pl
from jax.experimental.pallas import tpu as pltpu
from jax.experimental.pallas import tpu_sc as plsc

B = 16384
D = 64
L = 16

_info = plsc.get_sparse_core_info()
NC, NS = _info.num_cores, _info.num_subcores
NW = NC * NS
BPW = B // NW
NCHUNK = BPW // 128
NGROUP = BPW // L


def _bpr_body(users_hbm, pos_hbm, neg_hbm, uemb_hbm, iemb_hbm,
              rui_hbm, ruj_hbm, loss_hbm,
              uidx_v, pidx_v, nidx_v, urows_v, prows_v, nrows_v,
              rui_v, ruj_v, loss_v, sem):
    wid = lax.axis_index("s") * NC + lax.axis_index("c")
    pltpu.sync_copy(users_hbm.at[pl.ds(wid * NCHUNK, NCHUNK)], uidx_v)
    pltpu.sync_copy(pos_hbm.at[pl.ds(wid * NCHUNK, NCHUNK)], pidx_v)
    pltpu.sync_copy(neg_hbm.at[pl.ds(wid * NCHUNK, NCHUNK)], nidx_v)

    copies = []
    for j in range(NCHUNK):
        dst = pl.ds(j * 128, 128)
        copies.append(pltpu.async_copy(uemb_hbm.at[uidx_v.at[j]],
                                       urows_v.at[dst], sem))
        copies.append(pltpu.async_copy(iemb_hbm.at[pidx_v.at[j]],
                                       prows_v.at[dst], sem))
        copies.append(pltpu.async_copy(iemb_hbm.at[nidx_v.at[j]],
                                       nrows_v.at[dst], sem))
    for c in copies:
        c.wait()

    iota = lax.broadcasted_iota(jnp.int32, (L,), 0)
    zero = jnp.zeros((L,), jnp.float32)

    def group(g, carry):
        l1, l2, l3 = carry
        rowv = g * L + iota
        rui_a = zero
        rui_b = zero
        ruj_a = zero
        ruj_b = zero
        for d in range(D):
            colv = iota ^ d
            iu = plsc.load_gather(urows_v, [rowv, colv])
            ip = plsc.load_gather(prows_v, [rowv, colv])
            iv = plsc.load_gather(nrows_v, [rowv, colv])
            if d % 2 == 0:
                rui_a = rui_a + iu * ip
                ruj_a = ruj_a + iu * iv
            else:
                rui_b = rui_b + iu * ip
                ruj_b = ruj_b + iu * iv
            l1 = l1 + iu * iu
            l2 = l2 + ip * ip
            l3 = l3 + iv * iv
        rui_v[pl.ds(g * L, L)] = rui_a + rui_b
        ruj_v[pl.ds(g * L, L)] = ruj_a + ruj_b
        return (l1, l2, l3)

    l1, l2, l3 = lax.fori_loop(0, NGROUP, group, (zero, zero, zero))
    loss_v[...] = l1 + l2 + l3

    pltpu.sync_copy(rui_v, rui_hbm.at[pl.ds(wid * BPW, BPW)])
    pltpu.sync_copy(ruj_v, ruj_hbm.at[pl.ds(wid * BPW, BPW)])
    pltpu.sync_copy(loss_v, loss_hbm.at[wid])


@jax.jit
def _bpr_sc(users, pos_items, neg_items, user_emb, item_emb):
    mesh = plsc.VectorSubcoreMesh(core_axis_name="c", subcore_axis_name="s")
    k = functools.partial(
        pl.kernel,
        mesh=mesh,
        compiler_params=pltpu.CompilerParams(needs_layout_passes=False,
                                              use_tc_tiling_on_sc=False),
        out_type=[
            jax.ShapeDtypeStruct((B,), jnp.float32),
            jax.ShapeDtypeStruct((B,), jnp.float32),
            jax.ShapeDtypeStruct((NW, L), jnp.float32),
        ],
        scratch_types=[
            pltpu.VMEM((NCHUNK, 128), jnp.int32),
            pltpu.VMEM((NCHUNK, 128), jnp.int32),
            pltpu.VMEM((NCHUNK, 128), jnp.int32),
            pltpu.VMEM((BPW, D), jnp.float32),
            pltpu.VMEM((BPW, D), jnp.float32),
            pltpu.VMEM((BPW, D), jnp.float32),
            pltpu.VMEM((BPW,), jnp.float32),
            pltpu.VMEM((BPW,), jnp.float32),
            pltpu.VMEM((L,), jnp.float32),
            pltpu.SemaphoreType.DMA,
        ],
    )(_bpr_body)
    u2 = users.reshape(NW * NCHUNK, 128).astype(jnp.int32)
    p2 = pos_items.reshape(NW * NCHUNK, 128).astype(jnp.int32)
    n2 = neg_items.reshape(NW * NCHUNK, 128).astype(jnp.int32)
    rui, ruj, loss_parts = k(u2, p2, n2, user_emb, item_emb)
    return (rui.reshape(B, 1), ruj.reshape(B, 1), jnp.sum(loss_parts))


def kernel(users, pos_items, neg_items, user_emb, item_emb):
    return _bpr_sc(users, pos_items, neg_items, user_emb, item_emb)


# speedup vs baseline: 1.3347x; 1.0343x over previous
import functools

import jax
import jax.numpy as jnp
from jax import lax
from jax.experimental import pallas as pl
from jax.experimental.pallas import tpu as pltpu
from jax.experimental.pallas import tpu_sc as plsc

N_ROWS = 100000
B = 16384
D = 64
W = 128
L = 16

_info = plsc.get_sparse_core_info()
NC, NS = _info.num_cores, _info.num_subcores
NW = NC * NS
BPW = B // NW
NROUND = 2
RB = BPW // NROUND
NGROUP = RB // L


def _bpr_body(users_hbm, pos_hbm, neg_hbm, uemb_hbm, iemb_hbm,
              rui_hbm, ruj_hbm, loss_hbm,
              uidx_v, pidx_v, nidx_v,
              urows_v, prows_v, nrows_v,
              rui_v, ruj_v, loss_v, sem):
    wid = lax.axis_index("s") * NC + lax.axis_index("c")
    base = wid * BPW
    pltpu.sync_copy(users_hbm.at[pl.ds(base, BPW)], uidx_v)
    pltpu.sync_copy(pos_hbm.at[pl.ds(base, BPW)], pidx_v)
    pltpu.sync_copy(neg_hbm.at[pl.ds(base, BPW)], nidx_v)

    iota = lax.broadcasted_iota(jnp.int32, (L,), 0)
    zero = jnp.zeros((L,), jnp.float32)

    def round_body(r, carry):
        l1, l2, l3 = carry
        copies = []
        for j in range(RB // 128):
            src = pl.ds(r * RB + j * 128, 128)
            dst = pl.ds(j * 128, 128)
            copies.append(pltpu.async_copy(uemb_hbm.at[uidx_v.at[src]],
                                           urows_v.at[dst], sem))
            copies.append(pltpu.async_copy(iemb_hbm.at[pidx_v.at[src]],
                                           prows_v.at[dst], sem))
            copies.append(pltpu.async_copy(iemb_hbm.at[nidx_v.at[src]],
                                           nrows_v.at[dst], sem))
        for c in copies:
            c.wait()

        def group(g, carry2):
            l1, l2, l3 = carry2
            rowv = g * L + iota
            out = pl.ds(r * RB + g * L, L)
            rui_a = zero
            rui_b = zero
            ruj_a = zero
            ruj_b = zero
            for d in range(D):
                colv = iota ^ d
                iu = plsc.load_gather(urows_v, [rowv, colv])
                ip = plsc.load_gather(prows_v, [rowv, colv])
                iv = plsc.load_gather(nrows_v, [rowv, colv])
                if d % 2 == 0:
                    rui_a = rui_a + iu * ip
                    ruj_a = ruj_a + iu * iv
                else:
                    rui_b = rui_b + iu * ip
                    ruj_b = ruj_b + iu * iv
                l1 = l1 + iu * iu
                l2 = l2 + ip * ip
                l3 = l3 + iv * iv
            rui_v[out] = rui_a + rui_b
            ruj_v[out] = ruj_a + ruj_b
            return (l1, l2, l3)

        return lax.fori_loop(0, NGROUP, group, (l1, l2, l3))

    carry = (zero, zero, zero)
    for r in range(NROUND):
        carry = round_body(r, carry)
    l1, l2, l3 = carry
    loss_v[...] = l1 + l2 + l3

    pltpu.sync_copy(rui_v, rui_hbm.at[pl.ds(base, BPW)])
    pltpu.sync_copy(ruj_v, ruj_hbm.at[pl.ds(base, BPW)])
    pltpu.sync_copy(loss_v, loss_hbm.at[wid])


@jax.jit
def _bpr_sc(users, pos_items, neg_items, user_emb, item_emb):
    mesh = plsc.VectorSubcoreMesh(core_axis_name="c", subcore_axis_name="s")
    k = functools.partial(
        pl.kernel,
        mesh=mesh,
        compiler_params=pltpu.CompilerParams(needs_layout_passes=False,
                                             use_tc_tiling_on_sc=False),
        out_type=[
            jax.ShapeDtypeStruct((B,), jnp.float32),
            jax.ShapeDtypeStruct((B,), jnp.float32),
            jax.ShapeDtypeStruct((NW, L), jnp.float32),
        ],
        scratch_types=[
            pltpu.VMEM((BPW,), jnp.int32),
            pltpu.VMEM((BPW,), jnp.int32),
            pltpu.VMEM((BPW,), jnp.int32),
            pltpu.VMEM((RB, W), jnp.float32),
            pltpu.VMEM((RB, W), jnp.float32),
            pltpu.VMEM((RB, W), jnp.float32),
            pltpu.VMEM((BPW,), jnp.float32),
            pltpu.VMEM((BPW,), jnp.float32),
            pltpu.VMEM((L,), jnp.float32),
            pltpu.SemaphoreType.DMA,
        ],
    )(_bpr_body)
    upad = jnp.pad(user_emb, ((0, 0), (0, W - D)))
    ipad = jnp.pad(item_emb, ((0, 0), (0, W - D)))
    rui, ruj, loss_parts = k(users.astype(jnp.int32), pos_items.astype(jnp.int32),
                             neg_items.astype(jnp.int32), upad, ipad)
    return (rui.reshape(B, 1), ruj.reshape(B, 1), jnp.sum(loss_parts))


def kernel(users, pos_items, neg_items, user_emb, item_emb):
    return _bpr_sc(users, pos_items, neg_items, user_emb, item_emb)
